# Initial kernel scaffold; baseline (speedup 1.0000x reference)
import jax
import jax.numpy as jnp
from jax.experimental import pallas as pl


def kernel(token_ids, embedding_table):
    raise NotImplementedError("write your pallas kernel here")


# speedup vs baseline: 2.2846x; 2.2846x over previous
import functools
import math

import jax
import jax.numpy as jnp
from jax import lax
from jax.experimental import pallas as pl
from jax.experimental.pallas import tpu as pltpu
from jax.experimental.pallas import tpu_sc as plsc

D_MODEL = 512
MAX_SEQ = 200
VOCAB = 76
VOCAB_PAD = 80
BATCH = 1024

_info = plsc.get_sparse_core_info()
_NUM_CORES = _info.num_cores
_NUM_SUBCORES = _info.num_subcores
_NUM_WORKERS = _NUM_CORES * _NUM_SUBCORES
_LANES = _info.num_lanes

N_ROWS = BATCH * MAX_SEQ
ROWS_PER_W = N_ROWS // _NUM_WORKERS
CHUNK = 64
N_CHUNKS = ROWS_PER_W // CHUNK


def _fuse_body(table_ref, out_ref):
    p = pl.program_id(0)
    ji = lax.broadcasted_iota(jnp.int32, (1, D_MODEL), 1)
    even = ((ji >> 1) << 1).astype(jnp.float32)
    inv_den = jnp.exp(even * (-math.log(10000.0) / D_MODEL))
    arg = p.astype(jnp.float32) * inv_den
    pe = jnp.where((ji & 1) == 0, jnp.sin(arg), jnp.cos(arg))
    out_ref[...] = (table_ref[...] * math.sqrt(float(D_MODEL)) + pe)[None]


_build_fused = pl.pallas_call(
    _fuse_body,
    grid=(MAX_SEQ,),
    in_specs=[pl.BlockSpec((VOCAB_PAD, D_MODEL), lambda p: (0, 0))],
    out_specs=pl.BlockSpec((1, VOCAB_PAD, D_MODEL), lambda p: (p, 0, 0)),
    out_shape=jax.ShapeDtypeStruct((MAX_SEQ, VOCAB_PAD, D_MODEL), jnp.float32),
)

_mesh = plsc.VectorSubcoreMesh(core_axis_name="c", subcore_axis_name="s")


@functools.partial(
    pl.kernel,
    out_type=jax.ShapeDtypeStruct((N_ROWS, D_MODEL), jnp.float32),
    mesh=_mesh,
    scratch_types=[
        pltpu.VMEM((ROWS_PER_W,), jnp.int32),
        pltpu.VMEM((ROWS_PER_W,), jnp.int32),
        pltpu.VMEM((CHUNK, D_MODEL), jnp.float32),
        pltpu.SemaphoreType.DMA,
    ],
)
def _gather_kernel(tok_hbm, fused_hbm, out_hbm, tok_v, idx_v, buf_v, gsem):
    wid = lax.axis_index("s") * _NUM_CORES + lax.axis_index("c")
    base = wid * ROWS_PER_W
    pltpu.sync_copy(tok_hbm.at[pl.ds(base, ROWS_PER_W)], tok_v)

    lanes = lax.iota(jnp.int32, _LANES)

    def idx_body(j, carry):
        o = j * _LANES
        tok = tok_v[pl.ds(o, _LANES)]
        pos = jnp.remainder(o + lanes, MAX_SEQ)
        idx_v[pl.ds(o, _LANES)] = pos * VOCAB_PAD + tok
        return carry

    lax.fori_loop(0, ROWS_PER_W // _LANES, idx_body, 0)

    def chunk_body(c, carry):
        r0 = c * CHUNK
        pltpu.async_copy(
            fused_hbm.at[idx_v.at[pl.ds(r0, CHUNK)]], buf_v, gsem
        ).wait()
        pltpu.sync_copy(buf_v, out_hbm.at[pl.ds(base + r0, CHUNK)])
        return carry

    lax.fori_loop(0, N_CHUNKS, chunk_body, 0)


def kernel(token_ids, embedding_table):
    tok_flat = token_ids.reshape(-1).astype(jnp.int32)
    table_pad = jnp.pad(embedding_table, ((0, VOCAB_PAD - VOCAB), (0, 0)))
    fused = _build_fused(table_pad).reshape(MAX_SEQ * VOCAB_PAD, D_MODEL)
    out = _gather_kernel(tok_flat, fused)
    return out.reshape(BATCH, MAX_SEQ, D_MODEL)


# speedup vs baseline: 2.6680x; 1.1678x over previous
import functools
import math

import jax
import jax.numpy as jnp
from jax import lax
from jax.experimental import pallas as pl
from jax.experimental.pallas import tpu as pltpu
from jax.experimental.pallas import tpu_sc as plsc

D_MODEL = 512
MAX_SEQ = 200
VOCAB = 76
VOCAB_PAD = 80
BATCH = 1024

_info = plsc.get_sparse_core_info()
_NUM_CORES = _info.num_cores
_NUM_SUBCORES = _info.num_subcores
_NUM_WORKERS = _NUM_CORES * _NUM_SUBCORES
_LANES = _info.num_lanes

N_ROWS = BATCH * MAX_SEQ
ROWS_PER_W = N_ROWS // _NUM_WORKERS
CHUNK = 40
N_CHUNKS = ROWS_PER_W // CHUNK
NBUF = 4


def _fuse_body(table_ref, out_ref):
    p = pl.program_id(0)
    ji = lax.broadcasted_iota(jnp.int32, (1, D_MODEL), 1)
    even = ((ji >> 1) << 1).astype(jnp.float32)
    inv_den = jnp.exp(even * (-math.log(10000.0) / D_MODEL))
    arg = p.astype(jnp.float32) * inv_den
    pe = jnp.where((ji & 1) == 0, jnp.sin(arg), jnp.cos(arg))
    out_ref[...] = (table_ref[...] * math.sqrt(float(D_MODEL)) + pe)[None]


_build_fused = pl.pallas_call(
    _fuse_body,
    grid=(MAX_SEQ,),
    in_specs=[pl.BlockSpec((VOCAB_PAD, D_MODEL), lambda p: (0, 0))],
    out_specs=pl.BlockSpec((1, VOCAB_PAD, D_MODEL), lambda p: (p, 0, 0)),
    out_shape=jax.ShapeDtypeStruct((MAX_SEQ, VOCAB_PAD, D_MODEL), jnp.float32),
)

_mesh = plsc.VectorSubcoreMesh(core_axis_name="c", subcore_axis_name="s")


@functools.partial(
    pl.kernel,
    out_type=jax.ShapeDtypeStruct((N_ROWS, D_MODEL), jnp.float32),
    mesh=_mesh,
    scratch_types=[
        pltpu.VMEM((ROWS_PER_W,), jnp.int32),
        pltpu.VMEM((ROWS_PER_W,), jnp.int32),
        [pltpu.VMEM((CHUNK, D_MODEL), jnp.float32) for _ in range(NBUF)],
        [pltpu.SemaphoreType.DMA for _ in range(NBUF)],
        [pltpu.SemaphoreType.DMA for _ in range(NBUF)],
    ],
)
def _gather_kernel(tok_hbm, fused_hbm, out_hbm, tok_v, idx_v, bufs, gsems, ssems):
    wid = lax.axis_index("s") * _NUM_CORES + lax.axis_index("c")
    base = wid * ROWS_PER_W
    pltpu.sync_copy(tok_hbm.at[pl.ds(base, ROWS_PER_W)], tok_v)

    lanes = lax.iota(jnp.int32, _LANES)

    def idx_body(j, carry):
        o = j * _LANES
        tok = tok_v[pl.ds(o, _LANES)]
        pos = jnp.remainder(o + lanes, MAX_SEQ)
        idx_v[pl.ds(o, _LANES)] = pos * VOCAB_PAD + tok
        return carry

    lax.fori_loop(0, ROWS_PER_W // _LANES, idx_body, 0)

    def fire_gather(c, b):
        pltpu.async_copy(
            fused_hbm.at[idx_v.at[pl.ds(c * CHUNK, CHUNK)]], bufs[b], gsems[b]
        )

    def wait_gather(b):
        pltpu.make_async_copy(
            out_hbm.at[pl.ds(base, CHUNK)], bufs[b], gsems[b]
        ).wait()

    def fire_scatter(c, b):
        pltpu.async_copy(
            bufs[b], out_hbm.at[pl.ds(base + c * CHUNK, CHUNK)], ssems[b]
        )

    def wait_scatter(b):
        pltpu.make_async_copy(
            bufs[b], out_hbm.at[pl.ds(base, CHUNK)], ssems[b]
        ).wait()

    fire_gather(0, 0)
    fire_gather(1, 1)
    for c in (0, 1):
        wait_gather(c)
        fire_scatter(c, c)
        fire_gather(c + 2, c + 2)

    def chunk_body(g, carry):
        for k in range(NBUF):
            c = 2 + g * NBUF + k
            b = (2 + k) % NBUF
            b2 = k
            wait_gather(b)
            fire_scatter(c, b)
            wait_scatter(b2)
            fire_gather(c + 2, b2)
        return carry

    lax.fori_loop(0, (N_CHUNKS - 4) // NBUF, chunk_body, 0)

    for c in (N_CHUNKS - 2, N_CHUNKS - 1):
        b = c % NBUF
        wait_gather(b)
        fire_scatter(c, b)
    for b in range(NBUF):
        wait_scatter(b)


def kernel(token_ids, embedding_table):
    tok_flat = token_ids.reshape(-1).astype(jnp.int32)
    table_pad = jnp.pad(embedding_table, ((0, VOCAB_PAD - VOCAB), (0, 0)))
    fused = _build_fused(table_pad).reshape(MAX_SEQ * VOCAB_PAD, D_MODEL)
    out = _gather_kernel(tok_flat, fused)
    return out.reshape(BATCH, MAX_SEQ, D_MODEL)


# speedup vs baseline: 3.1481x; 1.1799x over previous
import functools
import math

import jax
import jax.numpy as jnp
from jax import lax
from jax.experimental import pallas as pl
from jax.experimental.pallas import tpu as pltpu
from jax.experimental.pallas import tpu_sc as plsc

D_MODEL = 512
MAX_SEQ = 200
VOCAB = 76
VOCAB_PAD = 80
BATCH = 1024

_info = plsc.get_sparse_core_info()
_NUM_CORES = _info.num_cores
_NUM_SUBCORES = _info.num_subcores
_NUM_WORKERS = _NUM_CORES * _NUM_SUBCORES
_LANES = _info.num_lanes

N_ROWS = BATCH * MAX_SEQ
ROWS_PER_W = N_ROWS // _NUM_WORKERS
CHUNK = 40
N_CHUNKS = ROWS_PER_W // CHUNK
NBUF = 4


def _positional_encoding():
    even_i = jnp.arange(0, D_MODEL, 2, dtype=jnp.float32)
    denominator = jnp.power(10000.0, even_i / D_MODEL)
    position = jnp.arange(0, MAX_SEQ, 1, dtype=jnp.float32).reshape(MAX_SEQ, 1)
    even_pe = jnp.sin(position / denominator)
    odd_pe = jnp.cos(position / denominator)
    return jnp.stack([even_pe, odd_pe], axis=2).reshape(MAX_SEQ, D_MODEL)


def _fuse_body(table_ref, pe_ref, out_ref):
    out_ref[...] = (
        table_ref[...] * math.sqrt(float(D_MODEL)) + pe_ref[...][:, None, :]
    )


_build_fused = pl.pallas_call(
    _fuse_body,
    out_shape=jax.ShapeDtypeStruct((MAX_SEQ, VOCAB_PAD, D_MODEL), jnp.float32),
)

_mesh = plsc.VectorSubcoreMesh(core_axis_name="c", subcore_axis_name="s")


@functools.partial(
    pl.kernel,
    out_type=jax.ShapeDtypeStruct((N_ROWS, D_MODEL), jnp.float32),
    mesh=_mesh,
    scratch_types=[
        pltpu.VMEM((ROWS_PER_W,), jnp.int32),
        pltpu.VMEM((ROWS_PER_W,), jnp.int32),
        [pltpu.VMEM((CHUNK, D_MODEL), jnp.float32) for _ in range(NBUF)],
        [pltpu.SemaphoreType.DMA for _ in range(NBUF)],
        [pltpu.SemaphoreType.DMA for _ in range(NBUF)],
    ],
)
def _gather_kernel(tok_hbm, fused_hbm, out_hbm, tok_v, idx_v, bufs, gsems, ssems):
    wid = lax.axis_index("s") * _NUM_CORES + lax.axis_index("c")
    base = wid * ROWS_PER_W
    pltpu.sync_copy(tok_hbm.at[pl.ds(base, ROWS_PER_W)], tok_v)

    lanes = lax.iota(jnp.int32, _LANES)

    def idx_body(j, carry):
        o = j * _LANES
        tok = tok_v[pl.ds(o, _LANES)]
        pos = jnp.remainder(o + lanes, MAX_SEQ)
        idx_v[pl.ds(o, _LANES)] = pos * VOCAB_PAD + tok
        return carry

    lax.fori_loop(0, ROWS_PER_W // _LANES, idx_body, 0)

    def fire_gather(c, b):
        pltpu.async_copy(
            fused_hbm.at[idx_v.at[pl.ds(c * CHUNK, CHUNK)]], bufs[b], gsems[b]
        )

    def wait_gather(b):
        pltpu.make_async_copy(
            out_hbm.at[pl.ds(base, CHUNK)], bufs[b], gsems[b]
        ).wait()

    def fire_scatter(c, b):
        pltpu.async_copy(
            bufs[b], out_hbm.at[pl.ds(base + c * CHUNK, CHUNK)], ssems[b]
        )

    def wait_scatter(b):
        pltpu.make_async_copy(
            bufs[b], out_hbm.at[pl.ds(base, CHUNK)], ssems[b]
        ).wait()

    fire_gather(0, 0)
    fire_gather(1, 1)
    for c in (0, 1):
        wait_gather(c)
        fire_scatter(c, c)
        fire_gather(c + 2, c + 2)

    def chunk_body(g, carry):
        for k in range(NBUF):
            c = 2 + g * NBUF + k
            b = (2 + k) % NBUF
            b2 = k
            wait_gather(b)
            fire_scatter(c, b)
            wait_scatter(b2)
            fire_gather(c + 2, b2)
        return carry

    lax.fori_loop(0, (N_CHUNKS - 4) // NBUF, chunk_body, 0)

    for c in (N_CHUNKS - 2, N_CHUNKS - 1):
        b = c % NBUF
        wait_gather(b)
        fire_scatter(c, b)
    for b in range(NBUF):
        wait_scatter(b)


def kernel(token_ids, embedding_table):
    tok_flat = token_ids.reshape(-1).astype(jnp.int32)
    table_pad = jnp.pad(embedding_table, ((0, VOCAB_PAD - VOCAB), (0, 0)))
    fused = _build_fused(table_pad, _positional_encoding()).reshape(
        MAX_SEQ * VOCAB_PAD, D_MODEL
    )
    out = _gather_kernel(tok_flat, fused)
    return out.reshape(BATCH, MAX_SEQ, D_MODEL)
